# baseline (speedup 1.0000x reference)
import functools

import jax
import jax.numpy as jnp
from jax import lax
from jax.experimental import pallas as pl
from jax.experimental.pallas import tpu as pltpu
from jax.experimental.pallas import tpu_sc as plsc

DIM = 32
IDX_PER_DMA = 128
CHUNK = 512
K = CHUNK // IDX_PER_DMA
LANES = 16


def _gather_kernel(seq, batch):
    n_chunks = seq
    assert n_chunks % 2 == 0
    n_pairs = n_chunks // 2
    mesh = plsc.VectorSubcoreMesh(core_axis_name="c", subcore_axis_name="s")
    nc = plsc.get_sparse_core_info().num_cores

    @functools.partial(
        pl.kernel,
        mesh=mesh,
        out_type=jax.ShapeDtypeStruct((seq, DIM, batch), jnp.float32),
        compiler_params=pltpu.CompilerParams(
            use_tc_tiling_on_sc=False, needs_layout_passes=False
        ),
        scratch_types=[
            pltpu.VMEM((seq, CHUNK), jnp.int32),
            pltpu.VMEM((CHUNK, DIM), jnp.float32),
            pltpu.VMEM((CHUNK, DIM), jnp.float32),
            pltpu.VMEM((DIM, CHUNK), jnp.float32),
            pltpu.VMEM((DIM, CHUNK), jnp.float32),
            pltpu.SemaphoreType.DMA,
            pltpu.SemaphoreType.DMA,
            pltpu.SemaphoreType.DMA,
            pltpu.SemaphoreType.DMA,
            pltpu.SemaphoreType.DMA,
        ],
    )
    def k(table_hbm, xt_hbm, out_hbm, xbuf, rows0, rows1, t0, t1,
          xsem, gsem0, gsem1, osem0, osem1):
        wid = lax.axis_index("s") * nc + lax.axis_index("c")
        b0 = pl.multiple_of(wid * CHUNK, CHUNK)
        lane = lax.iota(jnp.int32, LANES)
        diag_cols = [(lane + d0) & (DIM - 1) for d0 in range(DIM)]

        pltpu.async_copy(xt_hbm.at[:, pl.ds(b0, CHUNK)], xbuf, xsem).wait()

        def fire_gathers(s, rowsbuf, sem):
            for j in range(K):
                pltpu.async_copy(
                    table_hbm.at[xbuf.at[s, pl.ds(j * IDX_PER_DMA, IDX_PER_DMA)]],
                    rowsbuf.at[pl.ds(j * IDX_PER_DMA, IDX_PER_DMA)],
                    sem,
                )

        def wait_gathers(rowsbuf, sem):
            pltpu.make_async_copy(
                out_hbm.at[0, :, pl.ds(0, CHUNK)], rowsbuf, sem
            ).wait()

        def transpose(rowsbuf, tbuf):
            def g_body(g, _):
                rows16 = lane + g * LANES
                for d0 in range(DIM):
                    cols = diag_cols[d0]
                    vals = plsc.load_gather(rowsbuf, [rows16, cols])
                    plsc.store_scatter(tbuf, [cols, rows16], vals)
                return 0

            lax.fori_loop(0, CHUNK // LANES, g_body, 0, unroll=4)

        def fire_out(s, tbuf, sem):
            pltpu.async_copy(tbuf, out_hbm.at[s, :, pl.ds(b0, CHUNK)], sem)

        def wait_out(tbuf, sem):
            pltpu.make_async_copy(
                tbuf, out_hbm.at[0, :, pl.ds(0, CHUNK)], sem
            ).wait()

        fire_gathers(0, rows0, gsem0)
        fire_gathers(1, rows1, gsem1)

        wait_gathers(rows0, gsem0)
        transpose(rows0, t0)
        fire_out(0, t0, osem0)
        fire_gathers(2, rows0, gsem0)
        wait_gathers(rows1, gsem1)
        transpose(rows1, t1)
        fire_out(1, t1, osem1)
        fire_gathers(3, rows1, gsem1)

        def body(t, _):
            a = 2 * t
            wait_gathers(rows0, gsem0)
            wait_out(t0, osem0)
            transpose(rows0, t0)
            fire_out(a, t0, osem0)
            fire_gathers(a + 2, rows0, gsem0)
            wait_gathers(rows1, gsem1)
            wait_out(t1, osem1)
            transpose(rows1, t1)
            fire_out(a + 1, t1, osem1)
            fire_gathers(a + 3, rows1, gsem1)
            return 0

        lax.fori_loop(1, n_pairs - 1, body, 0, unroll=False)

        last = n_chunks - 2
        wait_gathers(rows0, gsem0)
        wait_out(t0, osem0)
        transpose(rows0, t0)
        fire_out(last, t0, osem0)
        wait_gathers(rows1, gsem1)
        wait_out(t1, osem1)
        transpose(rows1, t1)
        fire_out(last + 1, t1, osem1)
        wait_out(t0, osem0)
        wait_out(t1, osem1)

    return k


def kernel(x, y, embedding):
    b, s = x.shape
    out_p = _gather_kernel(s, b)(embedding, x.T.astype(jnp.int32))
    return jnp.transpose(out_p, (2, 0, 1))


# speedup vs baseline: 1.0071x; 1.0071x over previous
import functools

import jax
import jax.numpy as jnp
from jax import lax
from jax.experimental import pallas as pl
from jax.experimental.pallas import tpu as pltpu
from jax.experimental.pallas import tpu_sc as plsc

DIM = 32
IDX_PER_DMA = 128
CHUNK = 512
K = CHUNK // IDX_PER_DMA
LANES = 16


def _gather_kernel(seq, batch):
    n_chunks = seq
    assert n_chunks % 2 == 0
    n_pairs = n_chunks // 2
    mesh = plsc.VectorSubcoreMesh(core_axis_name="c", subcore_axis_name="s")
    nc = plsc.get_sparse_core_info().num_cores

    @functools.partial(
        pl.kernel,
        mesh=mesh,
        out_type=jax.ShapeDtypeStruct((seq, DIM, batch), jnp.float32),
        compiler_params=pltpu.CompilerParams(
            use_tc_tiling_on_sc=False, needs_layout_passes=False
        ),
        scratch_types=[
            pltpu.VMEM((seq, CHUNK), jnp.int32),
            pltpu.VMEM((CHUNK, DIM), jnp.float32),
            pltpu.VMEM((CHUNK, DIM), jnp.float32),
            pltpu.VMEM((DIM, CHUNK), jnp.float32),
            pltpu.VMEM((DIM, CHUNK), jnp.float32),
            pltpu.SemaphoreType.DMA,
            pltpu.SemaphoreType.DMA,
            pltpu.SemaphoreType.DMA,
            pltpu.SemaphoreType.DMA,
            pltpu.SemaphoreType.DMA,
        ],
    )
    def k(table_hbm, xt_hbm, out_hbm, xbuf, rows0, rows1, t0, t1,
          xsem, gsem0, gsem1, osem0, osem1):
        wid = lax.axis_index("s") * nc + lax.axis_index("c")
        b0 = pl.multiple_of(wid * CHUNK, CHUNK)
        lane = lax.iota(jnp.int32, LANES)
        diag_cols = [(lane + d0) & (DIM - 1) for d0 in range(DIM)]

        pltpu.async_copy(xt_hbm.at[:, pl.ds(b0, CHUNK)], xbuf, xsem).wait()

        def fire_gathers(s, rowsbuf, sem):
            for j in range(K):
                pltpu.async_copy(
                    table_hbm.at[xbuf.at[s, pl.ds(j * IDX_PER_DMA, IDX_PER_DMA)]],
                    rowsbuf.at[pl.ds(j * IDX_PER_DMA, IDX_PER_DMA)],
                    sem,
                )

        def wait_gathers(rowsbuf, sem):
            pltpu.make_async_copy(
                out_hbm.at[0, :, pl.ds(0, CHUNK)], rowsbuf, sem
            ).wait()

        def transpose(rowsbuf, tbuf):
            def g_body(g, _):
                rows16 = lane + g * LANES
                for d0 in range(DIM):
                    cols = diag_cols[d0]
                    vals = plsc.load_gather(rowsbuf, [rows16, cols])
                    plsc.store_scatter(tbuf, [cols, rows16], vals)
                return 0

            lax.fori_loop(0, CHUNK // LANES, g_body, 0, unroll=2)

        def fire_out(s, tbuf, sem):
            pltpu.async_copy(tbuf, out_hbm.at[s, :, pl.ds(b0, CHUNK)], sem)

        def wait_out(tbuf, sem):
            pltpu.make_async_copy(
                tbuf, out_hbm.at[0, :, pl.ds(0, CHUNK)], sem
            ).wait()

        fire_gathers(0, rows0, gsem0)
        fire_gathers(1, rows1, gsem1)

        wait_gathers(rows0, gsem0)
        transpose(rows0, t0)
        fire_out(0, t0, osem0)
        fire_gathers(2, rows0, gsem0)
        wait_gathers(rows1, gsem1)
        transpose(rows1, t1)
        fire_out(1, t1, osem1)
        fire_gathers(3, rows1, gsem1)

        def body(t, _):
            a = 2 * t
            wait_gathers(rows0, gsem0)
            wait_out(t0, osem0)
            transpose(rows0, t0)
            fire_out(a, t0, osem0)
            fire_gathers(a + 2, rows0, gsem0)
            wait_gathers(rows1, gsem1)
            wait_out(t1, osem1)
            transpose(rows1, t1)
            fire_out(a + 1, t1, osem1)
            fire_gathers(a + 3, rows1, gsem1)
            return 0

        lax.fori_loop(1, n_pairs - 1, body, 0, unroll=False)

        last = n_chunks - 2
        wait_gathers(rows0, gsem0)
        wait_out(t0, osem0)
        transpose(rows0, t0)
        fire_out(last, t0, osem0)
        wait_gathers(rows1, gsem1)
        wait_out(t1, osem1)
        transpose(rows1, t1)
        fire_out(last + 1, t1, osem1)
        wait_out(t0, osem0)
        wait_out(t1, osem1)

    return k


def kernel(x, y, embedding):
    b, s = x.shape
    out_p = _gather_kernel(s, b)(embedding, x.T.astype(jnp.int32))
    return jnp.transpose(out_p, (2, 0, 1))
